# baseline (speedup 1.0000x reference)
import functools

import jax
import jax.numpy as jnp
from jax import lax
from jax.experimental import pallas as pl
from jax.experimental.pallas import tpu as pltpu
from jax.experimental.pallas import tpu_sc as plsc

D = 64
SCALE = 8.0
L = 16
C = 160


@functools.cache
def _make_gather(B: int):
    info = plsc.get_sparse_core_info()
    NW = info.num_cores * info.num_subcores
    b_per_w = B // NW
    nchunks = b_per_w // C
    assert B % NW == 0 and b_per_w % C == 0 and nchunks % 2 == 0

    mesh = plsc.VectorSubcoreMesh(core_axis_name="c", subcore_axis_name="s")

    @functools.partial(
        pl.kernel,
        mesh=mesh,
        out_type=jax.ShapeDtypeStruct((B // 2, 2 * D), jnp.float32),
        scratch_types=[
            pltpu.VMEM((b_per_w,), jnp.int32),
            [pltpu.VMEM((C, D), jnp.float32)] * 2,
            [pltpu.VMEM((C // 2, 2 * D), jnp.float32)] * 2,
            [pltpu.SemaphoreType.DMA] * 2,
            [pltpu.SemaphoreType.DMA] * 2,
        ],
        compiler_params=pltpu.CompilerParams(use_tc_tiling_on_sc=True),
    )
    def k(table, idx, out, idx_v, gbuf, obuf, gsem, osem):
        wid = lax.axis_index("s") * info.num_cores + lax.axis_index("c")
        base = wid * b_per_w
        pltpu.sync_copy(idx.at[pl.ds(base, b_per_w)], idx_v)

        def start_gather(c, p):
            def grp(g, _):
                v = idx_v[pl.ds(c * C + g * L, L)]
                for lane in range(L):
                    s = v[lane]
                    pltpu.async_copy(
                        table.at[s >> 3, s & 7],
                        gbuf[p].at[g * L + lane],
                        gsem[p],
                    )
                return 0

            lax.fori_loop(0, C // L, grp, 0)

        def finish_chunk(d, q, wait_out):
            pltpu.make_async_copy(
                out.at[pl.ds(0, C // 2)], obuf[q], gsem[q]
            ).wait()
            pltpu.make_async_copy(
                out.at[pl.ds(0, C // 2)], obuf[q], gsem[q]
            ).wait()
            if wait_out:
                pltpu.make_async_copy(
                    out.at[pl.ds(0, C // 2)], obuf[q], osem[q]
                ).wait()

            def sc(j, _):
                for p2 in range(2):
                    for i in range(D // L):
                        obuf[q][j, pl.ds(p2 * D + i * L, L)] = (
                            gbuf[q][2 * j + p2, pl.ds(i * L, L)] * SCALE
                        )
                return 0

            lax.fori_loop(0, C // 2, sc, 0)
            pltpu.async_copy(
                obuf[q],
                out.at[pl.ds(wid * (b_per_w // 2) + d * (C // 2), C // 2)],
                osem[q],
            )

        start_gather(0, 0)
        start_gather(1, 1)
        finish_chunk(0, 0, False)
        start_gather(2, 0)
        finish_chunk(1, 1, False)
        start_gather(3, 1)

        def body(c2, _):
            d = 2 * c2
            finish_chunk(d, 0, True)
            start_gather(d + 2, 0)
            finish_chunk(d + 1, 1, True)
            start_gather(d + 3, 1)
            return 0

        lax.fori_loop(1, nchunks // 2 - 1, body, 0)

        finish_chunk(nchunks - 2, 0, True)
        finish_chunk(nchunks - 1, 1, True)
        pltpu.make_async_copy(out.at[pl.ds(0, C // 2)], obuf[0], osem[0]).wait()
        pltpu.make_async_copy(out.at[pl.ds(0, C // 2)], obuf[1], osem[1]).wait()

    return k


def kernel(x, input_embedding_table):
    B = x.shape[0] * x.shape[1]
    V = input_embedding_table.shape[0]
    t3 = input_embedding_table.reshape(V // 8, 8, D)
    idx = x.reshape(B).astype(jnp.int32)
    out = _make_gather(B)(t3, idx)
    return out.reshape(x.shape[0], x.shape[1], D)


# speedup vs baseline: 1.0756x; 1.0756x over previous
import functools

import jax
import jax.numpy as jnp
from jax import lax
from jax.experimental import pallas as pl
from jax.experimental.pallas import tpu as pltpu
from jax.experimental.pallas import tpu_sc as plsc

D = 64
SCALE = 8.0
L = 16
C = 400


@functools.cache
def _make_gather(B: int):
    info = plsc.get_sparse_core_info()
    NW = info.num_cores * info.num_subcores
    b_per_w = B // NW
    nchunks = b_per_w // C
    assert B % NW == 0 and b_per_w % C == 0 and nchunks % 2 == 0

    mesh = plsc.VectorSubcoreMesh(core_axis_name="c", subcore_axis_name="s")

    @functools.partial(
        pl.kernel,
        mesh=mesh,
        out_type=jax.ShapeDtypeStruct((B, D), jnp.float32),
        scratch_types=[
            pltpu.VMEM((b_per_w,), jnp.int32),
            [pltpu.VMEM((C, D), jnp.float32)] * 2,
            [pltpu.SemaphoreType.DMA] * 2,
            [pltpu.SemaphoreType.DMA] * 2,
        ],
        compiler_params=pltpu.CompilerParams(use_tc_tiling_on_sc=True),
    )
    def k(table, idx, out, idx_v, gbuf, gsem, osem):
        wid = lax.axis_index("s") * info.num_cores + lax.axis_index("c")
        base = wid * b_per_w
        pltpu.sync_copy(idx.at[pl.ds(base, b_per_w)], idx_v)

        def start_gather(c, p, wait_out):
            if wait_out:
                pltpu.make_async_copy(
                    out.at[pl.ds(0, C)], gbuf[p], osem[p]
                ).wait()

            def grp(g, _):
                v = idx_v[pl.ds(c * C + g * L, L)]
                for lane in range(L):
                    s = v[lane]
                    pltpu.async_copy(
                        table.at[s >> 3, s & 7],
                        gbuf[p].at[g * L + lane],
                        gsem[p],
                    )
                return 0

            lax.fori_loop(0, C // L, grp, 0)

        def finish_chunk(d, q):
            pltpu.make_async_copy(
                out.at[pl.ds(0, C)], gbuf[q], gsem[q]
            ).wait()

            def sc(j, _):
                for i in range(D // L):
                    sl = pl.ds(i * L, L)
                    gbuf[q][j, sl] = gbuf[q][j, sl] * SCALE
                return 0

            lax.fori_loop(0, C, sc, 0)
            pltpu.async_copy(gbuf[q], out.at[pl.ds(base + d * C, C)], osem[q])

        start_gather(0, 0, False)
        start_gather(1, 1, False)
        finish_chunk(0, 0)
        start_gather(2, 0, True)
        finish_chunk(1, 1)
        start_gather(3, 1, True)

        def body(c2, _):
            d = 2 * c2
            finish_chunk(d, 0)
            start_gather(d + 2, 0, True)
            finish_chunk(d + 1, 1)
            start_gather(d + 3, 1, True)
            return 0

        lax.fori_loop(1, nchunks // 2 - 1, body, 0)

        finish_chunk(nchunks - 2, 0)
        finish_chunk(nchunks - 1, 1)
        pltpu.make_async_copy(out.at[pl.ds(0, C)], gbuf[0], osem[0]).wait()
        pltpu.make_async_copy(out.at[pl.ds(0, C)], gbuf[1], osem[1]).wait()

    return k


def kernel(x, input_embedding_table):
    B = x.shape[0] * x.shape[1]
    V = input_embedding_table.shape[0]
    t3 = input_embedding_table.reshape(V // 8, 8, D)
    idx = x.reshape(B).astype(jnp.int32)
    out = _make_gather(B)(t3, idx)
    return out.reshape(x.shape[0], x.shape[1], D)
